# baseline (speedup 1.0000x reference)
import jax
import jax.numpy as jnp
from jax import lax
from jax.experimental import pallas as pl
from jax.experimental.pallas import tpu as pltpu
from jax.experimental.pallas import tpu_sc as plsc

NC, NS = 2, 16
NW = NC * NS
NCELLS = 200000
NFEAT = 128
C = 128
NCH_FULL = NCELLS // C
TAIL = NCELLS - NCH_FULL * C
ROW = 1 + 2 + 6 + 3 * NFEAT


def _process_chunk(cb, n, refs):
    u_r, tri_r, cc_r, vp_r, out_r, idx_v, vf_v, cc_v, vp_v, row_v, sem = refs
    lane = lax.broadcasted_iota(jnp.int32, (16,), 0)

    for k in range(3):
        pltpu.sync_copy(tri_r.at[pl.ds(k * NCELLS + cb, n)],
                        idx_v.at[pl.ds(k * C, n)])
    cps = [
        pltpu.async_copy(u_r.at[idx_v.at[pl.ds(k * C, n)]],
                         vf_v.at[pl.ds(k * C, n)], sem)
        for k in range(3)
    ]
    pltpu.sync_copy(cc_r.at[pl.ds(cb, n)], cc_v.at[pl.ds(0, n)])
    pltpu.sync_copy(vp_r.at[pl.ds(cb, n)], vp_v.at[pl.ds(0, n)])
    for g in range(2 * n // 16):
        e = g * 16 + lane
        vals = plsc.load_gather(cc_v, [e // 2, e % 2])
        plsc.store_scatter(row_v, [e // 2, 1 + e % 2], vals)
    for g in range(6 * n // 16):
        e = g * 16 + lane
        vals = plsc.load_gather(vp_v, [e // 6, e % 6])
        plsc.store_scatter(row_v, [e // 6, 3 + e % 6], vals)
    for k in range(3):
        cps[k].wait()

    def move_row(i, carry):
        for k in range(3):
            for j in range(NFEAT // 16):
                row_v[i, pl.ds(9 + k * NFEAT + j * 16, 16)] = (
                    vf_v[k * C + i, pl.ds(j * 16, 16)])
        return carry

    lax.fori_loop(0, n, move_row, 0)

    pltpu.sync_copy(row_v.at[pl.ds(0, n)], out_r.at[pl.ds(cb, n)])


def _body(u_r, tri_r, cc_r, vp_r, t_r, out_r,
          idx_v, vf_v, cc_v, vp_v, row_v, tval_v, sem):
    wid = lax.axis_index("s") * NC + lax.axis_index("c")
    lane = lax.broadcasted_iota(jnp.int32, (16,), 0)

    pltpu.sync_copy(t_r, tval_v)
    t_splat = tval_v[...]

    for g in range(C // 16):
        plsc.store_scatter(row_v, [g * 16 + lane, jnp.zeros((16,), jnp.int32)],
                           t_splat)

    refs = (u_r, tri_r, cc_r, vp_r, out_r, idx_v, vf_v, cc_v, vp_v, row_v,
            sem)

    nj = (NCH_FULL - 1) // NW + 1

    def loop_body(j, carry):
        ch = wid + j * NW

        @pl.when(ch < NCH_FULL)
        def _():
            _process_chunk(ch * C, C, refs)
        return carry

    lax.fori_loop(0, nj, loop_body, 0)

    @pl.when(wid == NW - 1)
    def _():
        _process_chunk(NCH_FULL * C, TAIL, refs)


@jax.jit
def _assemble(u2, tri_t, cc2, vp6, t1):
    mesh = plsc.VectorSubcoreMesh(core_axis_name="c", subcore_axis_name="s",
                                  num_cores=NC, num_subcores=NS)
    k = pl.kernel(
        _body,
        out_type=jax.ShapeDtypeStruct((NCELLS, ROW), jnp.float32),
        mesh=mesh,
        scratch_types=[
            pltpu.VMEM((3 * C,), jnp.int32),
            pltpu.VMEM((3 * C, NFEAT), jnp.float32),
            pltpu.VMEM((C, 2), jnp.float32),
            pltpu.VMEM((C, 6), jnp.float32),
            pltpu.VMEM((C, ROW), jnp.float32),
            pltpu.VMEM((16,), jnp.float32),
            pltpu.SemaphoreType.DMA,
        ],
        compiler_params=pltpu.CompilerParams(use_tc_tiling_on_sc=False,
                                             needs_layout_passes=False),
    )
    return k(u2, tri_t, cc2, vp6, t1)


def kernel(u, t, cell_centers, cell_local_vertex_pos, triangulation):
    u2 = u.reshape(u.shape[1], u.shape[2])
    tri_t = triangulation.astype(jnp.int32).T.reshape(-1)
    cc2 = cell_centers
    vp6 = cell_local_vertex_pos.reshape(NCELLS, 6)
    t1 = jnp.broadcast_to(t.reshape(1), (16,))
    out = _assemble(u2, tri_t, cc2, vp6, t1)
    return out[None]
